# baseline (speedup 1.0000x reference)
import functools

import jax
import jax.numpy as jnp
from jax import lax
from jax.experimental import pallas as pl
from jax.experimental.pallas import tpu as pltpu
from jax.experimental.pallas import tpu_sc as plsc

NC = 2
NS = 16
NW = NC * NS
L = 16
WIDE = 128
CHUNK = 128


@functools.partial(jax.jit, static_argnames=("B", "D"))
def _gmf(uidx, iidx, ut_wide, it_wide, w_flat, b_vec, *, B, D):
    bpw = B // NW
    n_chunks = bpw // CHUNK
    n_groups = CHUNK // L
    rpw = WIDE // D
    shift = rpw.bit_length() - 1

    mesh = plsc.VectorSubcoreMesh(
        core_axis_name="c", subcore_axis_name="s",
        num_cores=NC, num_subcores=NS,
    )

    @functools.partial(
        pl.kernel,
        out_type=jax.ShapeDtypeStruct((B,), jnp.float32),
        mesh=mesh,
        compiler_params=pltpu.CompilerParams(needs_layout_passes=False),
        scratch_types=[
            pltpu.VMEM((n_chunks, CHUNK), jnp.int32),
            pltpu.VMEM((n_chunks, CHUNK), jnp.int32),
            pltpu.VMEM((n_chunks, CHUNK), jnp.int32),
            pltpu.VMEM((n_chunks, CHUNK), jnp.int32),
            pltpu.VMEM((CHUNK, WIDE), jnp.float32),
            pltpu.VMEM((CHUNK, WIDE), jnp.float32),
            pltpu.VMEM((CHUNK, WIDE), jnp.float32),
            pltpu.VMEM((CHUNK, WIDE), jnp.float32),
            pltpu.VMEM((D,), jnp.float32),
            pltpu.VMEM((L,), jnp.float32),
            pltpu.VMEM((bpw,), jnp.float32),
            pltpu.SemaphoreType.DMA,
            pltpu.SemaphoreType.DMA,
        ],
    )
    def k(uidx_hbm, iidx_hbm, ut_hbm, it_hbm, w_hbm, b_hbm, out_hbm,
          uidx_v, iidx_v, urow_v, irow_v, ur0, ur1, ir0, ir1,
          w_v, b_v, out_v, sem0, sem1):
        wid = lax.axis_index("s") * NC + lax.axis_index("c")
        pltpu.sync_copy(uidx_hbm.at[wid], uidx_v)
        pltpu.sync_copy(iidx_hbm.at[wid], iidx_v)
        pltpu.sync_copy(w_hbm, w_v)
        pltpu.sync_copy(b_hbm, b_v)

        for c in range(n_chunks):
            for s in range(CHUNK // L):
                sl = pl.ds(s * L, L)
                urow_v[c, sl] = uidx_v[c, sl] >> shift
                irow_v[c, sl] = iidx_v[c, sl] >> shift

        ubufs, ibufs, sems = [ur0, ur1], [ir0, ir1], [sem0, sem1]

        def start(c):
            slot = c % 2
            return (
                pltpu.async_copy(ut_hbm.at[urow_v.at[c]], ubufs[slot], sems[slot]),
                pltpu.async_copy(it_hbm.at[irow_v.at[c]], ibufs[slot], sems[slot]),
            )

        pending = {0: start(0)}
        lane = lax.iota(jnp.int32, L)
        bias = b_v[...]
        w_regs = [w_v[pl.ds(i * L, L)] for i in range(D // L)]

        for c in range(n_chunks):
            if c + 1 < n_chunks:
                pending[c + 1] = start(c + 1)
            for cp in pending.pop(c):
                cp.wait()
            ub, ib = ubufs[c % 2], ibufs[c % 2]

            def group(g, carry):
                rows = lane + g * L
                uoff = (uidx_v[c, pl.ds(g * L, L)] & (rpw - 1)) * D
                ioff = (iidx_v[c, pl.ds(g * L, L)] & (rpw - 1)) * D
                acc = bias
                for d in range(D):
                    u_col = plsc.load_gather(ub, [rows, uoff + d])
                    i_col = plsc.load_gather(ib, [rows, ioff + d])
                    acc = acc + u_col * i_col * w_regs[d // L][d % L]
                rating = 1.0 / (1.0 + jnp.exp(-acc))
                out_v[pl.ds(c * CHUNK + g * L, L)] = rating
                return carry

            lax.fori_loop(0, n_groups, group, 0)

        pltpu.sync_copy(out_v, out_hbm.at[pl.ds(wid * bpw, bpw)])

    return k(uidx, iidx, ut_wide, it_wide, w_flat, b_vec)


def kernel(user_indices, item_indices, user_table, item_table, W, b):
    B = user_indices.shape[0]
    D = user_table.shape[1]
    uidx = user_indices.astype(jnp.int32).reshape(NW, (B // NW) // CHUNK, CHUNK)
    iidx = item_indices.astype(jnp.int32).reshape(NW, (B // NW) // CHUNK, CHUNK)
    ut_wide = user_table.reshape(-1, WIDE)
    it_wide = item_table.reshape(-1, WIDE)
    w_flat = W.reshape(D).astype(jnp.float32)
    b_vec = jnp.broadcast_to(b.reshape(1), (L,)).astype(jnp.float32)
    out = _gmf(uidx, iidx, ut_wide, it_wide, w_flat, b_vec, B=B, D=D)
    return out.reshape(B, 1)


# speedup vs baseline: 1.0076x; 1.0076x over previous
import functools

import jax
import jax.numpy as jnp
from jax import lax
from jax.experimental import pallas as pl
from jax.experimental.pallas import tpu as pltpu
from jax.experimental.pallas import tpu_sc as plsc

NC = 2
NS = 16
NW = NC * NS
L = 16
CHUNK = 128


@functools.partial(jax.jit, static_argnames=("B", "D"))
def _gmf(uidx, iidx, user_table, item_table, w_flat, b_vec, *, B, D):
    bpw = B // NW
    n_chunks = bpw // CHUNK
    n_groups = CHUNK // L

    mesh = plsc.VectorSubcoreMesh(
        core_axis_name="c", subcore_axis_name="s",
        num_cores=NC, num_subcores=NS,
    )

    @functools.partial(
        pl.kernel,
        out_type=jax.ShapeDtypeStruct((B,), jnp.float32),
        mesh=mesh,
        compiler_params=pltpu.CompilerParams(
            needs_layout_passes=False, use_tc_tiling_on_sc=False),
        scratch_types=[
            pltpu.VMEM((n_chunks, CHUNK), jnp.int32),
            pltpu.VMEM((n_chunks, CHUNK), jnp.int32),
            pltpu.VMEM((bpw, D), jnp.float32),
            pltpu.VMEM((bpw, D), jnp.float32),
            pltpu.VMEM((D,), jnp.float32),
            pltpu.VMEM((L,), jnp.float32),
            pltpu.VMEM((bpw,), jnp.float32),
        ] + [pltpu.SemaphoreType.DMA] * (bpw // CHUNK),
    )
    def k(uidx_hbm, iidx_hbm, ut_hbm, it_hbm, w_hbm, b_hbm, out_hbm,
          uidx_v, iidx_v, urows_v, irows_v, w_v, b_v, out_v, *sems):
        wid = lax.axis_index("s") * NC + lax.axis_index("c")
        pltpu.sync_copy(uidx_hbm.at[wid], uidx_v)
        pltpu.sync_copy(iidx_hbm.at[wid], iidx_v)
        pltpu.sync_copy(w_hbm, w_v)
        pltpu.sync_copy(b_hbm, b_v)

        pending = []
        for c in range(n_chunks):
            rows = pl.ds(c * CHUNK, CHUNK)
            pending.append((
                pltpu.async_copy(ut_hbm.at[uidx_v.at[c]], urows_v.at[rows], sems[c]),
                pltpu.async_copy(it_hbm.at[iidx_v.at[c]], irows_v.at[rows], sems[c]),
            ))

        lane = lax.iota(jnp.int32, L)
        bias = b_v[...]
        w_regs = [w_v[pl.ds(i * L, L)] for i in range(D // L)]

        for c in range(n_chunks):
            for cp in pending[c]:
                cp.wait()

            def group(g, carry):
                rows = lane + (c * CHUNK + g * L)
                acc = bias
                for d in range(D):
                    col = jnp.full((L,), d, jnp.int32)
                    u_col = plsc.load_gather(urows_v, [rows, col])
                    i_col = plsc.load_gather(irows_v, [rows, col])
                    acc = acc + u_col * i_col * w_regs[d // L][d % L]
                rating = 1.0 / (1.0 + jnp.exp(-acc))
                out_v[pl.ds(c * CHUNK + g * L, L)] = rating
                return carry

            lax.fori_loop(0, n_groups, group, 0)

        pltpu.sync_copy(out_v, out_hbm.at[pl.ds(wid * bpw, bpw)])

    return k(uidx, iidx, user_table, item_table, w_flat, b_vec)


def kernel(user_indices, item_indices, user_table, item_table, W, b):
    B = user_indices.shape[0]
    D = user_table.shape[1]
    uidx = user_indices.astype(jnp.int32).reshape(NW, (B // NW) // CHUNK, CHUNK)
    iidx = item_indices.astype(jnp.int32).reshape(NW, (B // NW) // CHUNK, CHUNK)
    w_flat = W.reshape(D).astype(jnp.float32)
    b_vec = jnp.broadcast_to(b.reshape(1), (L,)).astype(jnp.float32)
    out = _gmf(uidx, iidx, user_table, item_table, w_flat, b_vec, B=B, D=D)
    return out.reshape(B, 1)
